# baseline (speedup 1.0000x reference)
import functools

import jax
import jax.numpy as jnp
from jax.experimental import pallas as pl
from jax.experimental.pallas import tpu as pltpu

N_E = 8192
E_DIM = 256
HALF = 128
BETA = 0.25
ENTROPY_RATIO = 0.1
TEMPERATURE = 0.01
TOKENS = 8 * 24 * 24
TB = 128
GRID = TOKENS // TB


def _vq_kernel(zf_ref, w1_ref, w2_ref,
               idx_ref, zq_ref, vq_ref, ent_ref, d1n_ref, d2n_ref,
               e1_ref, e2_ref, e1n_ref, e2n_ref, aprob_ref):
    i = pl.program_id(0)

    @pl.when(i == 0)
    def _init():
        w1 = w1_ref[...]
        n1 = jnp.sqrt(jnp.sum(w1 * w1, axis=1, keepdims=True))
        e1 = w1 / jnp.maximum(n1, 1e-12)
        e1_ref[...] = e1
        w2 = w2_ref[...]
        n2 = jnp.sqrt(jnp.sum(w2 * w2, axis=1, keepdims=True))
        e2 = w2 / jnp.maximum(n2, 1e-12)
        e2_ref[...] = e2
        ones = jnp.ones((1, HALF), jnp.float32)
        e1n_ref[...] = jax.lax.dot_general(
            ones, e1 * e1, (((1,), (1,)), ((), ())),
            preferred_element_type=jnp.float32,
            precision=jax.lax.Precision.HIGHEST)
        e2n_ref[...] = jax.lax.dot_general(
            ones, e2 * e2, (((1,), (1,)), ((), ())),
            preferred_element_type=jnp.float32,
            precision=jax.lax.Precision.HIGHEST)
        aprob_ref[...] = jnp.zeros_like(aprob_ref)
        zero = jnp.zeros((1, 1), jnp.float32)
        vq_ref[...] = zero
        ent_ref[...] = zero
        d1n_ref[...] = zero
        d2n_ref[...] = zero

    zf = zf_ref[...]
    z1 = zf[:, :HALF]
    z2 = zf[:, HALF:]
    zn1 = z1 / jnp.maximum(jnp.sqrt(jnp.sum(z1 * z1, axis=1, keepdims=True)), 1e-12)
    zn2 = z2 / jnp.maximum(jnp.sqrt(jnp.sum(z2 * z2, axis=1, keepdims=True)), 1e-12)
    sz1 = jnp.sum(zn1 * zn1, axis=1, keepdims=True)
    sz2 = jnp.sum(zn2 * zn2, axis=1, keepdims=True)

    e1 = e1_ref[...]
    e2 = e2_ref[...]
    s1 = jax.lax.dot_general(zn1.astype(jnp.bfloat16), e1.astype(jnp.bfloat16),
                             (((1,), (1,)), ((), ())),
                             preferred_element_type=jnp.float32)
    s2 = jax.lax.dot_general(zn2.astype(jnp.bfloat16), e2.astype(jnp.bfloat16),
                             (((1,), (1,)), ((), ())),
                             preferred_element_type=jnp.float32)
    d1 = (sz1 + e1n_ref[...]) - 2.0 * s1
    d2 = (sz2 + e2n_ref[...]) - 2.0 * s2
    d = d1 + d2

    d1n_ref[...] += jnp.sum(d1 * d1).reshape(1, 1)
    d2n_ref[...] += jnp.sum(d2 * d2).reshape(1, 1)

    idx = jnp.argmin(d, axis=1)
    dmin = jnp.min(d, axis=1)
    idx_ref[...] = idx.reshape(TB, 1)
    vq_ref[...] += jnp.sum(dmin).reshape(1, 1)

    flat = (-d) / TEMPERATURE
    m = jnp.max(flat, axis=1, keepdims=True)
    shifted = flat - m
    ex = jnp.exp(shifted)
    zsum = jnp.sum(ex, axis=1, keepdims=True)
    p = ex / zsum
    row_plogp = jnp.sum(p * shifted, axis=1, keepdims=True) - jnp.log(zsum)
    ent_ref[...] += jnp.sum(row_plogp).reshape(1, 1)
    aprob_ref[...] += jnp.sum(p, axis=0, keepdims=True)

    onehot = (jax.lax.broadcasted_iota(jnp.int32, (TB, N_E), 1)
              == idx.reshape(TB, 1)).astype(jnp.float32)
    zq1 = jax.lax.dot_general(onehot, e1, (((1,), (0,)), ((), ())),
                              preferred_element_type=jnp.float32,
                              precision=jax.lax.Precision.HIGHEST)
    zq2 = jax.lax.dot_general(onehot, e2, (((1,), (0,)), ((), ())),
                              preferred_element_type=jnp.float32,
                              precision=jax.lax.Precision.HIGHEST)
    zq_ref[...] = jnp.concatenate([zq1, zq2], axis=1)

    @pl.when(i == GRID - 1)
    def _finalize():
        ntok = jnp.float32(TOKENS)
        vq_ref[...] = vq_ref[...] / (ntok * jnp.float32(E_DIM))
        sample_entropy = -(ent_ref[...] / ntok)
        avg = aprob_ref[...] / ntok
        avg_entropy = -jnp.sum(avg * jnp.log(avg + 1e-5)).reshape(1, 1)
        ent_ref[...] = ENTROPY_RATIO * (sample_entropy - avg_entropy)
        d1n_ref[...] = d1n_ref[...] / ntok
        d2n_ref[...] = d2n_ref[...] / ntok


@functools.partial(jax.jit, static_argnames=())
def _vq_call(zf, w1, w2):
    out_shapes = (
        jax.ShapeDtypeStruct((TOKENS, 1), jnp.int32),
        jax.ShapeDtypeStruct((TOKENS, E_DIM), jnp.float32),
        jax.ShapeDtypeStruct((1, 1), jnp.float32),
        jax.ShapeDtypeStruct((1, 1), jnp.float32),
        jax.ShapeDtypeStruct((1, 1), jnp.float32),
        jax.ShapeDtypeStruct((1, 1), jnp.float32),
    )
    scalar_spec = pl.BlockSpec((1, 1), lambda i: (0, 0))
    return pl.pallas_call(
        _vq_kernel,
        grid=(GRID,),
        in_specs=[
            pl.BlockSpec((TB, E_DIM), lambda i: (i, 0)),
            pl.BlockSpec((N_E, HALF), lambda i: (0, 0)),
            pl.BlockSpec((N_E, HALF), lambda i: (0, 0)),
        ],
        out_specs=(
            pl.BlockSpec((TB, 1), lambda i: (i, 0)),
            pl.BlockSpec((TB, E_DIM), lambda i: (i, 0)),
            scalar_spec, scalar_spec, scalar_spec, scalar_spec,
        ),
        out_shape=out_shapes,
        scratch_shapes=[
            pltpu.VMEM((N_E, HALF), jnp.float32),
            pltpu.VMEM((N_E, HALF), jnp.float32),
            pltpu.VMEM((1, N_E), jnp.float32),
            pltpu.VMEM((1, N_E), jnp.float32),
            pltpu.VMEM((1, N_E), jnp.float32),
        ],
    )(zf, w1, w2)


def kernel(z, W_vqkd, W_vqgan):
    b, c, h, w = z.shape
    zf = jnp.transpose(z, (0, 2, 3, 1)).reshape(-1, E_DIM)
    idx2, z_q, vq, ent, d1n, d2n = _vq_call(zf, W_vqkd, W_vqgan)
    indices = idx2.reshape(-1)
    out = jnp.transpose(z_q.reshape(b, h, w, c), (0, 3, 1, 2))
    vq_loss = vq[0, 0]
    commit_loss = BETA * vq_loss
    entropy_loss = ent[0, 0]
    return (out, vq_loss, commit_loss, entropy_loss,
            d1n[0, 0], d2n[0, 0], indices)


# speedup vs baseline: 1.9803x; 1.9803x over previous
import functools

import jax
import jax.numpy as jnp
from jax import lax
from jax.experimental import pallas as pl
from jax.experimental.pallas import tpu as pltpu
from jax.experimental.pallas import tpu_sc as plsc

N_E = 8192
E_DIM = 256
HALF = 128
BETA = 0.25
ENTROPY_RATIO = 0.1
INV_TEMP = 100.0
TOKENS = 8 * 24 * 24
TB = 128
GRID = TOKENS // TB

NW = 32
B_PER_W = TOKENS // NW


def _vq_kernel(zf_ref, w1_ref, w2_ref,
               idx_ref, emb_ref, vq_ref, ent_ref, d1n_ref, d2n_ref,
               e1b_ref, e2b_ref, e1n_ref, e2n_ref, aprob_ref):
    i = pl.program_id(0)

    @pl.when(i == 0)
    def _init():
        w1 = w1_ref[...]
        n1 = jnp.sqrt(jnp.sum(w1 * w1, axis=1, keepdims=True))
        e1 = w1 / jnp.maximum(n1, 1e-12)
        w2 = w2_ref[...]
        n2 = jnp.sqrt(jnp.sum(w2 * w2, axis=1, keepdims=True))
        e2 = w2 / jnp.maximum(n2, 1e-12)
        emb_ref[:, :HALF] = e1
        emb_ref[:, HALF:] = e2
        e1b_ref[...] = e1.astype(jnp.bfloat16)
        e2b_ref[...] = e2.astype(jnp.bfloat16)
        ones = jnp.ones((1, HALF), jnp.float32)
        e1n_ref[...] = jax.lax.dot_general(
            ones, e1 * e1, (((1,), (1,)), ((), ())),
            preferred_element_type=jnp.float32,
            precision=jax.lax.Precision.HIGHEST)
        e2n_ref[...] = jax.lax.dot_general(
            ones, e2 * e2, (((1,), (1,)), ((), ())),
            preferred_element_type=jnp.float32,
            precision=jax.lax.Precision.HIGHEST)
        aprob_ref[...] = jnp.zeros_like(aprob_ref)
        zero = jnp.zeros((1, 1), jnp.float32)
        vq_ref[...] = zero
        ent_ref[...] = zero
        d1n_ref[...] = zero
        d2n_ref[...] = zero

    zf = zf_ref[...]
    z1 = zf[:, :HALF]
    z2 = zf[:, HALF:]
    zn1 = z1 / jnp.maximum(jnp.sqrt(jnp.sum(z1 * z1, axis=1, keepdims=True)), 1e-12)
    zn2 = z2 / jnp.maximum(jnp.sqrt(jnp.sum(z2 * z2, axis=1, keepdims=True)), 1e-12)
    sz1 = jnp.sum(zn1 * zn1, axis=1, keepdims=True)
    sz2 = jnp.sum(zn2 * zn2, axis=1, keepdims=True)

    s1 = jax.lax.dot_general(zn1.astype(jnp.bfloat16), e1b_ref[...],
                             (((1,), (1,)), ((), ())),
                             preferred_element_type=jnp.float32)
    s2 = jax.lax.dot_general(zn2.astype(jnp.bfloat16), e2b_ref[...],
                             (((1,), (1,)), ((), ())),
                             preferred_element_type=jnp.float32)
    d1 = (sz1 + e1n_ref[...]) - 2.0 * s1
    d2 = (sz2 + e2n_ref[...]) - 2.0 * s2
    d = d1 + d2

    d1n_ref[...] += jnp.sum(d1 * d1).reshape(1, 1)
    d2n_ref[...] += jnp.sum(d2 * d2).reshape(1, 1)

    idx = jnp.argmin(d, axis=1)
    dmin = jnp.min(d, axis=1, keepdims=True)
    idx_ref[...] = idx.reshape(TB, 1)
    vq_ref[...] += jnp.sum(dmin).reshape(1, 1)

    shifted = (dmin - d) * INV_TEMP
    ex = jnp.exp(shifted)
    zsum = jnp.sum(ex, axis=1, keepdims=True)
    rz = 1.0 / zsum
    t = jnp.sum(ex * shifted, axis=1, keepdims=True)
    ent_ref[...] += jnp.sum(t * rz - jnp.log(zsum)).reshape(1, 1)
    aprob_ref[...] += jnp.sum(ex * rz, axis=0, keepdims=True)

    @pl.when(i == GRID - 1)
    def _finalize():
        ntok = jnp.float32(TOKENS)
        vq_ref[...] = vq_ref[...] / (ntok * jnp.float32(E_DIM))
        sample_entropy = -(ent_ref[...] / ntok)
        avg = aprob_ref[...] / ntok
        avg_entropy = -jnp.sum(avg * jnp.log(avg + 1e-5)).reshape(1, 1)
        ent_ref[...] = ENTROPY_RATIO * (sample_entropy - avg_entropy)
        d1n_ref[...] = d1n_ref[...] / ntok
        d2n_ref[...] = d2n_ref[...] / ntok


def _vq_call(zf, w1, w2):
    out_shapes = (
        jax.ShapeDtypeStruct((TOKENS, 1), jnp.int32),
        jax.ShapeDtypeStruct((N_E, E_DIM), jnp.float32),
        jax.ShapeDtypeStruct((1, 1), jnp.float32),
        jax.ShapeDtypeStruct((1, 1), jnp.float32),
        jax.ShapeDtypeStruct((1, 1), jnp.float32),
        jax.ShapeDtypeStruct((1, 1), jnp.float32),
    )
    scalar_spec = pl.BlockSpec((1, 1), lambda i: (0, 0))
    return pl.pallas_call(
        _vq_kernel,
        grid=(GRID,),
        in_specs=[
            pl.BlockSpec((TB, E_DIM), lambda i: (i, 0)),
            pl.BlockSpec((N_E, HALF), lambda i: (0, 0)),
            pl.BlockSpec((N_E, HALF), lambda i: (0, 0)),
        ],
        out_specs=(
            pl.BlockSpec((TB, 1), lambda i: (i, 0)),
            pl.BlockSpec((N_E, E_DIM), lambda i: (0, 0)),
            scalar_spec, scalar_spec, scalar_spec, scalar_spec,
        ),
        out_shape=out_shapes,
        scratch_shapes=[
            pltpu.VMEM((N_E, HALF), jnp.bfloat16),
            pltpu.VMEM((N_E, HALF), jnp.bfloat16),
            pltpu.VMEM((1, N_E), jnp.float32),
            pltpu.VMEM((1, N_E), jnp.float32),
            pltpu.VMEM((1, N_E), jnp.float32),
        ],
    )(zf, w1, w2)


@functools.lru_cache(maxsize=1)
def _sc_gather_fn():
    mesh = plsc.VectorSubcoreMesh(core_axis_name="c", subcore_axis_name="s")

    @functools.partial(
        pl.kernel,
        mesh=mesh,
        out_type=jax.ShapeDtypeStruct((TOKENS, E_DIM), jnp.float32),
        scratch_types=[
            pltpu.VMEM((B_PER_W,), jnp.int32),
            pltpu.VMEM((B_PER_W, E_DIM), jnp.float32),
            pltpu.SemaphoreType.DMA,
        ],
    )
    def _sc_gather(table_hbm, idx_hbm, out_hbm, idx_v, rows_v, sem):
        wid = lax.axis_index("s") * 2 + lax.axis_index("c")
        base = wid * B_PER_W
        pltpu.sync_copy(idx_hbm.at[pl.ds(base, B_PER_W)], idx_v)
        pltpu.async_copy(table_hbm.at[idx_v], rows_v, sem).wait()
        pltpu.sync_copy(rows_v, out_hbm.at[pl.ds(base, B_PER_W)])

    return _sc_gather


def kernel(z, W_vqkd, W_vqgan):
    b, c, h, w = z.shape
    zf = jnp.transpose(z, (0, 2, 3, 1)).reshape(-1, E_DIM)
    idx2, emb, vq, ent, d1n, d2n = _vq_call(zf, W_vqkd, W_vqgan)
    indices = idx2.reshape(-1)
    z_q = _sc_gather_fn()(emb, indices)
    out = jnp.transpose(z_q.reshape(b, h, w, c), (0, 3, 1, 2))
    vq_loss = vq[0, 0]
    commit_loss = BETA * vq_loss
    entropy_loss = ent[0, 0]
    return (out, vq_loss, commit_loss, entropy_loss,
            d1n[0, 0], d2n[0, 0], indices)


# speedup vs baseline: 2.1576x; 1.0895x over previous
import functools

import jax
import jax.numpy as jnp
from jax import lax
from jax.experimental import pallas as pl
from jax.experimental.pallas import tpu as pltpu
from jax.experimental.pallas import tpu_sc as plsc

N_E = 8192
E_DIM = 256
HALF = 128
BETA = 0.25
ENTROPY_RATIO = 0.1
INV_TEMP = 100.0
TOKENS = 8 * 24 * 24
TB = 128
GRID = TOKENS // TB

NW = 32
B_PER_W = TOKENS // NW


def _vq_kernel(zf_ref, w1_ref, w2_ref,
               idx_ref, emb_ref, vq_ref, ent_ref, d1n_ref, d2n_ref,
               e1b_ref, e2b_ref, e1n_ref, e2n_ref, aprob_ref, dsq_ref):
    i = pl.program_id(0)

    @pl.when(i == 0)
    def _init():
        w1 = w1_ref[...]
        n1 = jnp.sqrt(jnp.sum(w1 * w1, axis=1, keepdims=True))
        e1 = w1 / jnp.maximum(n1, 1e-12)
        w2 = w2_ref[...]
        n2 = jnp.sqrt(jnp.sum(w2 * w2, axis=1, keepdims=True))
        e2 = w2 / jnp.maximum(n2, 1e-12)
        emb_ref[:, :HALF] = e1
        emb_ref[:, HALF:] = e2
        e1b_ref[...] = e1.astype(jnp.bfloat16)
        e2b_ref[...] = e2.astype(jnp.bfloat16)
        ones = jnp.ones((1, HALF), jnp.float32)
        e1n_ref[...] = jax.lax.dot_general(
            ones, e1 * e1, (((1,), (1,)), ((), ())),
            preferred_element_type=jnp.float32,
            precision=jax.lax.Precision.HIGHEST)
        e2n_ref[...] = jax.lax.dot_general(
            ones, e2 * e2, (((1,), (1,)), ((), ())),
            preferred_element_type=jnp.float32,
            precision=jax.lax.Precision.HIGHEST)
        aprob_ref[...] = jnp.zeros_like(aprob_ref)
        dsq_ref[...] = jnp.zeros_like(dsq_ref)
        zero = jnp.zeros((1, 1), jnp.float32)
        vq_ref[...] = zero
        ent_ref[...] = zero

    zf = zf_ref[...]
    z1 = zf[:, :HALF]
    z2 = zf[:, HALF:]
    zn1 = z1 / jnp.maximum(jnp.sqrt(jnp.sum(z1 * z1, axis=1, keepdims=True)), 1e-12)
    zn2 = z2 / jnp.maximum(jnp.sqrt(jnp.sum(z2 * z2, axis=1, keepdims=True)), 1e-12)
    sz1 = jnp.sum(zn1 * zn1, axis=1, keepdims=True)
    sz2 = jnp.sum(zn2 * zn2, axis=1, keepdims=True)

    s1 = jax.lax.dot_general(zn1.astype(jnp.bfloat16), e1b_ref[...],
                             (((1,), (1,)), ((), ())),
                             preferred_element_type=jnp.float32)
    s2 = jax.lax.dot_general(zn2.astype(jnp.bfloat16), e2b_ref[...],
                             (((1,), (1,)), ((), ())),
                             preferred_element_type=jnp.float32)
    d1 = (sz1 + e1n_ref[...]) - 2.0 * s1
    d2 = (sz2 + e2n_ref[...]) - 2.0 * s2
    d = d1 + d2

    ones_row = jnp.ones((1, TB), jnp.float32)
    dsq_ref[...] += jnp.concatenate([
        jax.lax.dot_general(ones_row, d1 * d1, (((1,), (0,)), ((), ())),
                            preferred_element_type=jnp.float32),
        jax.lax.dot_general(ones_row, d2 * d2, (((1,), (0,)), ((), ())),
                            preferred_element_type=jnp.float32),
    ], axis=0)

    idx = jnp.argmin(d, axis=1)
    dmin = jnp.min(d, axis=1, keepdims=True)
    idx_ref[...] = idx.reshape(TB, 1)
    vq_ref[...] += jnp.sum(dmin).reshape(1, 1)

    shifted = (dmin - d) * INV_TEMP
    ex = jnp.exp(shifted)
    zsum = jnp.sum(ex, axis=1, keepdims=True)
    rz = 1.0 / zsum
    t = jnp.sum(ex * shifted, axis=1, keepdims=True)
    ent_ref[...] += jnp.sum(t * rz - jnp.log(zsum)).reshape(1, 1)
    aprob_ref[...] += jax.lax.dot_general(
        rz.reshape(1, TB), ex, (((1,), (0,)), ((), ())),
        preferred_element_type=jnp.float32)

    @pl.when(i == GRID - 1)
    def _finalize():
        ntok = jnp.float32(TOKENS)
        vq_ref[...] = vq_ref[...] / (ntok * jnp.float32(E_DIM))
        sample_entropy = -(ent_ref[...] / ntok)
        avg = aprob_ref[...] / ntok
        avg_entropy = -jnp.sum(avg * jnp.log(avg + 1e-5)).reshape(1, 1)
        ent_ref[...] = ENTROPY_RATIO * (sample_entropy - avg_entropy)
        dsq = dsq_ref[...]
        d1n_ref[...] = (jnp.sum(dsq[0:1, :]) / ntok).reshape(1, 1)
        d2n_ref[...] = (jnp.sum(dsq[1:2, :]) / ntok).reshape(1, 1)


def _vq_call(zf, w1, w2):
    out_shapes = (
        jax.ShapeDtypeStruct((TOKENS, 1), jnp.int32),
        jax.ShapeDtypeStruct((N_E, E_DIM), jnp.float32),
        jax.ShapeDtypeStruct((1, 1), jnp.float32),
        jax.ShapeDtypeStruct((1, 1), jnp.float32),
        jax.ShapeDtypeStruct((1, 1), jnp.float32),
        jax.ShapeDtypeStruct((1, 1), jnp.float32),
    )
    scalar_spec = pl.BlockSpec((1, 1), lambda i: (0, 0))
    return pl.pallas_call(
        _vq_kernel,
        grid=(GRID,),
        in_specs=[
            pl.BlockSpec((TB, E_DIM), lambda i: (i, 0)),
            pl.BlockSpec((N_E, HALF), lambda i: (0, 0)),
            pl.BlockSpec((N_E, HALF), lambda i: (0, 0)),
        ],
        out_specs=(
            pl.BlockSpec((TB, 1), lambda i: (i, 0)),
            pl.BlockSpec((N_E, E_DIM), lambda i: (0, 0)),
            scalar_spec, scalar_spec, scalar_spec, scalar_spec,
        ),
        out_shape=out_shapes,
        scratch_shapes=[
            pltpu.VMEM((N_E, HALF), jnp.bfloat16),
            pltpu.VMEM((N_E, HALF), jnp.bfloat16),
            pltpu.VMEM((1, N_E), jnp.float32),
            pltpu.VMEM((1, N_E), jnp.float32),
            pltpu.VMEM((1, N_E), jnp.float32),
            pltpu.VMEM((2, N_E), jnp.float32),
        ],
    )(zf, w1, w2)


@functools.lru_cache(maxsize=1)
def _sc_gather_fn():
    mesh = plsc.VectorSubcoreMesh(core_axis_name="c", subcore_axis_name="s")

    @functools.partial(
        pl.kernel,
        mesh=mesh,
        out_type=jax.ShapeDtypeStruct((TOKENS, E_DIM), jnp.float32),
        scratch_types=[
            pltpu.VMEM((B_PER_W,), jnp.int32),
            pltpu.VMEM((B_PER_W, E_DIM), jnp.float32),
            pltpu.SemaphoreType.DMA,
        ],
    )
    def _sc_gather(table_hbm, idx_hbm, out_hbm, idx_v, rows_v, sem):
        wid = lax.axis_index("s") * 2 + lax.axis_index("c")
        base = wid * B_PER_W
        pltpu.sync_copy(idx_hbm.at[pl.ds(base, B_PER_W)], idx_v)
        pltpu.async_copy(table_hbm.at[idx_v], rows_v, sem).wait()
        pltpu.sync_copy(rows_v, out_hbm.at[pl.ds(base, B_PER_W)])

    return _sc_gather


def kernel(z, W_vqkd, W_vqgan):
    b, c, h, w = z.shape
    zf = jnp.transpose(z, (0, 2, 3, 1)).reshape(-1, E_DIM)
    idx2, emb, vq, ent, d1n, d2n = _vq_call(zf, W_vqkd, W_vqgan)
    indices = idx2.reshape(-1)
    z_q = _sc_gather_fn()(emb, indices)
    out = jnp.transpose(z_q.reshape(b, h, w, c), (0, 3, 1, 2))
    vq_loss = vq[0, 0]
    commit_loss = BETA * vq_loss
    entropy_loss = ent[0, 0]
    return (out, vq_loss, commit_loss, entropy_loss,
            d1n[0, 0], d2n[0, 0], indices)
